# Initial kernel scaffold; baseline (speedup 1.0000x reference)
import functools

import jax
import jax.numpy as jnp
from jax import lax
from jax.experimental import pallas as pl
from jax.experimental.pallas import tpu as pltpu
from jax.experimental.pallas import tpu_sc as plsc

B, L, D = 4096, 50, 32
N = B * L
NC, NS = 2, 16
NW = NC * NS
PW = N // NW
CH = 128
CPG = 5
GR = CPG * CH
NG = PW // GR
K = PW // CH


def _sc_body(t_u, t_i, t_c, t_x, ix, out, idx_v, rows_v, sem):
    wid = lax.axis_index("s") * NC + lax.axis_index("c")
    base = wid * PW
    pltpu.sync_copy(ix.at[wid], idx_v)
    for f, tab in enumerate((t_u, t_i, t_c, t_x)):
        def grp(g, carry, f=f, tab=tab):
            cps = []
            for t in range(CPG):
                cps.append(pltpu.async_copy(
                    tab.at[idx_v.at[f, g * CPG + t]],
                    rows_v.at[pl.ds(t * CH, CH), :],
                    sem))
            for c in cps:
                c.wait()
            pltpu.sync_copy(rows_v, out.at[f, pl.ds(base + g * GR, GR), :])
            return carry
        lax.fori_loop(0, NG, grp, 0)


def _sc_gather(t_u, t_i, t_c, t_x, ix):
    mesh = plsc.VectorSubcoreMesh(core_axis_name="c", subcore_axis_name="s")
    fn = functools.partial(
        pl.kernel,
        mesh=mesh,
        out_type=jax.ShapeDtypeStruct((4, N, D), jnp.float32),
        scratch_types=[
            pltpu.VMEM((4, K, CH), jnp.int32),
            pltpu.VMEM((GR, D), jnp.float32),
            pltpu.SemaphoreType.DMA,
        ],
    )(_sc_body)
    return fn(t_u, t_i, t_c, t_x, ix)


BLK = 2048


def _mm_body(e_ref, w_ref, b_ref, o_ref):
    e = jnp.concatenate([e_ref[0], e_ref[1], e_ref[2], e_ref[3]], axis=1)
    o_ref[...] = (
        jnp.dot(e, w_ref[...], preferred_element_type=jnp.float32) + b_ref[...]
    )


def _project(e, W, b2):
    return pl.pallas_call(
        _mm_body,
        grid=(N // BLK,),
        in_specs=[
            pl.BlockSpec((4, BLK, D), lambda i: (0, i, 0)),
            pl.BlockSpec((4 * D, 128), lambda i: (0, 0)),
            pl.BlockSpec((1, 128), lambda i: (0, 0)),
        ],
        out_specs=pl.BlockSpec((BLK, 128), lambda i: (i, 0)),
        out_shape=jax.ShapeDtypeStruct((N, 128), jnp.float32),
    )(e, W, b2)


def kernel(user, item, category, interaction, emb_user, emb_item,
           emb_category, emb_interaction, W, b):
    ix = jnp.stack([
        user.reshape(-1).astype(jnp.int32),
        item.reshape(-1).astype(jnp.int32),
        category.reshape(-1).astype(jnp.int32),
        interaction.reshape(-1).astype(jnp.int32),
    ], axis=0)
    ix = ix.reshape(4, NW, K, CH).transpose(1, 0, 2, 3)
    e = _sc_gather(emb_user, emb_item, emb_category, emb_interaction, ix)
    out = _project(e, W, b.reshape(1, 128))
    return out.reshape(B, L, 128)


# speedup vs baseline: 2.4067x; 2.4067x over previous
import functools

import jax
import jax.numpy as jnp
from jax import lax
from jax.experimental import pallas as pl
from jax.experimental.pallas import tpu as pltpu
from jax.experimental.pallas import tpu_sc as plsc

B, L, D = 4096, 50, 32
N = B * L
NC, NS = 2, 16
NW = NC * NS
PW = N // NW
CH = 128
CPG = 5
GR = CPG * CH
NG = PW // GR
K = PW // CH


def _sc_body(t_u, t_i, t_c, t_x, ix, out, idx_v, rows_v, sem):
    wid = lax.axis_index("s") * NC + lax.axis_index("c")
    base = wid * PW
    pltpu.sync_copy(ix.at[wid], idx_v)
    for f, tab in enumerate((t_u, t_i, t_c, t_x)):
        def grp(g, carry, f=f, tab=tab):
            cps = []
            for t in range(CPG):
                cps.append(pltpu.async_copy(
                    tab.at[idx_v.at[f, g * CPG + t]],
                    rows_v.at[pl.ds(t * CH, CH), :],
                    sem))
            for c in cps:
                c.wait()
            pltpu.sync_copy(rows_v, out.at[f, pl.ds(base + g * GR, GR), :])
            return carry
        lax.fori_loop(0, NG, grp, 0)


def _sc_gather(t_u, t_i, t_c, t_x, ix):
    mesh = plsc.VectorSubcoreMesh(core_axis_name="c", subcore_axis_name="s")
    fn = functools.partial(
        pl.kernel,
        mesh=mesh,
        out_type=jax.ShapeDtypeStruct((4, N, D), jnp.float32),
        scratch_types=[
            pltpu.VMEM((4, K, CH), jnp.int32),
            pltpu.VMEM((GR, D), jnp.float32),
            pltpu.SemaphoreType.DMA,
        ],
        compiler_params=pltpu.CompilerParams(use_tc_tiling_on_sc=False),
    )(_sc_body)
    return fn(t_u, t_i, t_c, t_x, ix)


BLK = 2048


def _mm_body(e_ref, w_ref, b_ref, o_ref):
    e = jnp.concatenate([e_ref[0], e_ref[1], e_ref[2], e_ref[3]], axis=1)
    o_ref[...] = (
        jnp.dot(e, w_ref[...], preferred_element_type=jnp.float32) + b_ref[...]
    )


def _project(e, W, b2):
    return pl.pallas_call(
        _mm_body,
        grid=(N // BLK,),
        in_specs=[
            pl.BlockSpec((4, BLK, D), lambda i: (0, i, 0)),
            pl.BlockSpec((4 * D, 128), lambda i: (0, 0)),
            pl.BlockSpec((1, 128), lambda i: (0, 0)),
        ],
        out_specs=pl.BlockSpec((BLK, 128), lambda i: (i, 0)),
        out_shape=jax.ShapeDtypeStruct((N, 128), jnp.float32),
    )(e, W, b2)


def kernel(user, item, category, interaction, emb_user, emb_item,
           emb_category, emb_interaction, W, b):
    ix = jnp.stack([
        user.reshape(-1).astype(jnp.int32),
        item.reshape(-1).astype(jnp.int32),
        category.reshape(-1).astype(jnp.int32),
        interaction.reshape(-1).astype(jnp.int32),
    ], axis=0)
    ix = ix.reshape(4, NW, K, CH).transpose(1, 0, 2, 3)
    e = _sc_gather(emb_user, emb_item, emb_category, emb_interaction, ix)
    out = _project(e, W, b.reshape(1, 128))
    return out.reshape(B, L, 128)


# speedup vs baseline: 2.6957x; 1.1201x over previous
import functools

import jax
import jax.numpy as jnp
from jax import lax
from jax.experimental import pallas as pl
from jax.experimental.pallas import tpu as pltpu
from jax.experimental.pallas import tpu_sc as plsc

B, L, D = 4096, 50, 32
N = B * L
NC, NS = 2, 16
NW = NC * NS
PW = N // NW
CH = 128
CPG = 5
GR = CPG * CH
NG = PW // GR
K = PW // CH


def _sc_body(t_u, t_i, t_c, t_x, ix_u, ix_i, ix_c, ix_x, out,
             idx_u, idx_i, idx_c, idx_x, rows0, rows1, sem_g, sem_w0, sem_w1):
    wid = lax.axis_index("s") * NC + lax.axis_index("c")
    base = wid * PW
    for ix, idxv in ((ix_u, idx_u), (ix_i, idx_i), (ix_c, idx_c), (ix_x, idx_x)):
        pltpu.sync_copy(ix.at[wid], idxv)
    rows = (rows0, rows1)
    sems = (sem_w0, sem_w1)
    for f, (tab, idxv) in enumerate(
            ((t_u, idx_u), (t_i, idx_i), (t_c, idx_c), (t_x, idx_x))):
        def grp2(g2, carry, tab=tab, idxv=idxv, f=f):
            for par in range(2):
                g = g2 * 2 + par
                dst = out.at[pl.ds(base + g * GR, GR), pl.ds(f * D, D)]

                @pl.when(g2 > 0)
                def _(par=par, dst=dst):
                    pltpu.make_async_copy(rows[par], dst, sems[par]).wait()

                cps = [pltpu.async_copy(
                           tab.at[idxv.at[g * CPG + t]],
                           rows[par].at[pl.ds(t * CH, CH), :],
                           sem_g)
                       for t in range(CPG)]
                for c in cps:
                    c.wait()
                pltpu.async_copy(rows[par], dst, sems[par])
            return carry
        lax.fori_loop(0, NG // 2, grp2, 0)
        for par in range(2):
            g = NG - 2 + par
            dst = out.at[pl.ds(base + g * GR, GR), pl.ds(f * D, D)]
            pltpu.make_async_copy(rows[par], dst, sems[par]).wait()


def _sc_gather(t_u, t_i, t_c, t_x, ix_u, ix_i, ix_c, ix_x):
    mesh = plsc.VectorSubcoreMesh(core_axis_name="c", subcore_axis_name="s")
    fn = functools.partial(
        pl.kernel,
        mesh=mesh,
        out_type=jax.ShapeDtypeStruct((N, 4 * D), jnp.float32),
        scratch_types=[
            pltpu.VMEM((K, CH), jnp.int32),
            pltpu.VMEM((K, CH), jnp.int32),
            pltpu.VMEM((K, CH), jnp.int32),
            pltpu.VMEM((K, CH), jnp.int32),
            pltpu.VMEM((GR, D), jnp.float32),
            pltpu.VMEM((GR, D), jnp.float32),
            pltpu.SemaphoreType.DMA,
            pltpu.SemaphoreType.DMA,
            pltpu.SemaphoreType.DMA,
        ],
        compiler_params=pltpu.CompilerParams(use_tc_tiling_on_sc=False),
    )(_sc_body)
    return fn(t_u, t_i, t_c, t_x, ix_u, ix_i, ix_c, ix_x)


BLK = 4096


def _mm_body(e_ref, w_ref, b_ref, o_ref):
    o_ref[...] = (
        jnp.dot(e_ref[...], w_ref[...], preferred_element_type=jnp.float32)
        + b_ref[...]
    )


def _project(e, W, b2):
    return pl.pallas_call(
        _mm_body,
        grid=(N // BLK,),
        in_specs=[
            pl.BlockSpec((BLK, 4 * D), lambda i: (i, 0)),
            pl.BlockSpec((4 * D, 128), lambda i: (0, 0)),
            pl.BlockSpec((1, 128), lambda i: (0, 0)),
        ],
        out_specs=pl.BlockSpec((BLK, 128), lambda i: (i, 0)),
        out_shape=jax.ShapeDtypeStruct((N, 128), jnp.float32),
    )(e, W, b2)


def kernel(user, item, category, interaction, emb_user, emb_item,
           emb_category, emb_interaction, W, b):
    ixs = [a.reshape(NW, K, CH).astype(jnp.int32)
           for a in (user, item, category, interaction)]
    e = _sc_gather(emb_user, emb_item, emb_category, emb_interaction, *ixs)
    out = _project(e, W, b.reshape(1, 128))
    return out.reshape(B, L, 128)


# speedup vs baseline: 6.9367x; 2.5732x over previous
import functools

import jax
import jax.numpy as jnp
from jax import lax
from jax.experimental import pallas as pl
from jax.experimental.pallas import tpu as pltpu
from jax.experimental.pallas import tpu_sc as plsc

B, L, D = 4096, 50, 32
N = B * L
NC, NS = 2, 16
NW = NC * NS
PW = N // NW
CH = 128
CPG = 5
GR = CPG * CH
NG = PW // GR
K = PW // CH
V_CAT1 = 1001
V_INT = 3


def _sc_body(t_u, t_i, t_c, t_x, ix_u, ix_i, ix_c, ix_x, out,
             idx_u, idx_i, idx_c, idx_x, cat_v, int_v,
             rows0, rows1, sem_g, sem_w0, sem_w1):
    wid = lax.axis_index("s") * NC + lax.axis_index("c")
    base = wid * PW
    pltpu.sync_copy(ix_u.at[wid], idx_u)
    pltpu.sync_copy(ix_i.at[wid], idx_i)
    pltpu.sync_copy(ix_c.at[wid], idx_c)
    pltpu.sync_copy(ix_x.at[wid], idx_x)
    pltpu.sync_copy(t_c, cat_v)
    pltpu.sync_copy(t_x, int_v)
    rows = (rows0, rows1)
    sems = (sem_w0, sem_w1)

    for f, (tab, idxv) in enumerate(((t_u, idx_u), (t_i, idx_i))):
        def grp2(g2, carry, tab=tab, idxv=idxv, f=f):
            for par in range(2):
                g = g2 * 2 + par
                dst = out.at[pl.ds(base + g * GR, GR), pl.ds(f * D, D)]

                @pl.when(g2 > 0)
                def _(par=par, dst=dst):
                    pltpu.make_async_copy(rows[par], dst, sems[par]).wait()

                cps = [pltpu.async_copy(
                           tab.at[idxv.at[g * CPG + t]],
                           rows[par].at[pl.ds(t * CH, CH), :],
                           sem_g)
                       for t in range(CPG)]
                for c in cps:
                    c.wait()
                pltpu.async_copy(rows[par], dst, sems[par])
            return carry
        lax.fori_loop(0, NG // 2, grp2, 0)
        for par in range(2):
            g = NG - 2 + par
            dst = out.at[pl.ds(base + g * GR, GR), pl.ds(f * D, D)]
            pltpu.make_async_copy(rows[par], dst, sems[par]).wait()

    lanes = lax.iota(jnp.int32, 16)
    for f, (tab_v, idxv) in enumerate(((cat_v, idx_c), (int_v, idx_x))):
        def grp2l(g2, carry, tab_v=tab_v, idxv=idxv, f=f + 2):
            for par in range(2):
                g = g2 * 2 + par
                dst = out.at[pl.ds(base + g * GR, GR), pl.ds(f * D, D)]

                @pl.when(g2 > 0)
                def _(par=par, dst=dst):
                    pltpu.make_async_copy(rows[par], dst, sems[par]).wait()

                def step(s, c2, tab_v=tab_v, idxv=idxv, par=par, g=g):
                    rowi = idxv[pl.ds(g * GR + s * 16, 16)]
                    ri = s * 16 + lanes
                    for col in range(D):
                        ci = jnp.full((16,), col, jnp.int32)
                        vals = plsc.load_gather(tab_v, [rowi, ci])
                        plsc.store_scatter(rows[par], [ri, ci], vals)
                    return c2
                lax.fori_loop(0, GR // 16, step, 0)
                pltpu.async_copy(rows[par], dst, sems[par])
            return carry
        lax.fori_loop(0, NG // 2, grp2l, 0)
        for par in range(2):
            g = NG - 2 + par
            dst = out.at[pl.ds(base + g * GR, GR), pl.ds((f + 2) * D, D)]
            pltpu.make_async_copy(rows[par], dst, sems[par]).wait()


def _sc_gather(t_u, t_i, t_c, t_x, ix_u, ix_i, ix_c, ix_x):
    mesh = plsc.VectorSubcoreMesh(core_axis_name="c", subcore_axis_name="s")
    fn = functools.partial(
        pl.kernel,
        mesh=mesh,
        out_type=jax.ShapeDtypeStruct((N, 4 * D), jnp.float32),
        scratch_types=[
            pltpu.VMEM((K, CH), jnp.int32),
            pltpu.VMEM((K, CH), jnp.int32),
            pltpu.VMEM((PW,), jnp.int32),
            pltpu.VMEM((PW,), jnp.int32),
            pltpu.VMEM((V_CAT1, D), jnp.float32),
            pltpu.VMEM((V_INT, D), jnp.float32),
            pltpu.VMEM((GR, D), jnp.float32),
            pltpu.VMEM((GR, D), jnp.float32),
            pltpu.SemaphoreType.DMA,
            pltpu.SemaphoreType.DMA,
            pltpu.SemaphoreType.DMA,
        ],
        compiler_params=pltpu.CompilerParams(
            use_tc_tiling_on_sc=False, needs_layout_passes=False),
    )(_sc_body)
    return fn(t_u, t_i, t_c, t_x, ix_u, ix_i, ix_c, ix_x)


BLK = 4096


def _mm_body(e_ref, w_ref, b_ref, o_ref):
    o_ref[...] = (
        jnp.dot(e_ref[...], w_ref[...], preferred_element_type=jnp.float32)
        + b_ref[...]
    )


def _project(e, W, b2):
    return pl.pallas_call(
        _mm_body,
        grid=(N // BLK,),
        in_specs=[
            pl.BlockSpec((BLK, 4 * D), lambda i: (i, 0)),
            pl.BlockSpec((4 * D, 128), lambda i: (0, 0)),
            pl.BlockSpec((1, 128), lambda i: (0, 0)),
        ],
        out_specs=pl.BlockSpec((BLK, 128), lambda i: (i, 0)),
        out_shape=jax.ShapeDtypeStruct((N, 128), jnp.float32),
    )(e, W, b2)


def kernel(user, item, category, interaction, emb_user, emb_item,
           emb_category, emb_interaction, W, b):
    ix_u = user.T.reshape(NW, K, CH).astype(jnp.int32)
    ix_i = item.T.reshape(NW, K, CH).astype(jnp.int32)
    ix_c = category.T.reshape(NW, K * CH).astype(jnp.int32)
    ix_x = interaction.T.reshape(NW, K * CH).astype(jnp.int32)
    e = _sc_gather(emb_user, emb_item, emb_category, emb_interaction,
                   ix_u, ix_i, ix_c, ix_x)
    out = _project(e, W, b.reshape(1, 128))
    return jnp.transpose(out.reshape(L, B, 128), (1, 0, 2))


# speedup vs baseline: 8.6704x; 1.2499x over previous
import functools

import jax
import jax.numpy as jnp
from jax import lax
from jax.experimental import pallas as pl
from jax.experimental.pallas import tpu as pltpu
from jax.experimental.pallas import tpu_sc as plsc

B, L, D = 4096, 50, 32
N = B * L
NC, NS = 2, 16
NW = NC * NS
PW = N // NW
CH = 128
CPG = 5
GR = CPG * CH
NG = PW // GR
K = PW // CH
V_CAT1 = 1001
V_INT = 3


def _sc_body(t_u, t_i, t_c, t_x, ix_u, ix_i, ix_c, ix_x, out,
             idx_u, idx_i, idx_c, idx_x, cat_v, int_v,
             rows0, rows1, sem_g, sem_w0, sem_w1):
    wid = lax.axis_index("s") * NC + lax.axis_index("c")
    base = wid * PW
    pltpu.sync_copy(ix_u.at[wid], idx_u)
    pltpu.sync_copy(ix_i.at[wid], idx_i)
    pltpu.sync_copy(ix_c.at[wid], idx_c)
    pltpu.sync_copy(ix_x.at[wid], idx_x)
    pltpu.sync_copy(t_c, cat_v)
    pltpu.sync_copy(t_x, int_v)
    rows = (rows0, rows1)
    sems = (sem_w0, sem_w1)

    for f, (tab, idxv) in enumerate(((t_u, idx_u), (t_i, idx_i))):
        def grp2(g2, carry, tab=tab, idxv=idxv, f=f):
            for par in range(2):
                g = g2 * 2 + par
                dst = out.at[pl.ds(base + g * GR, GR), pl.ds(f * D, D)]

                @pl.when(g2 > 0)
                def _(par=par, dst=dst):
                    pltpu.make_async_copy(rows[par], dst, sems[par]).wait()

                cps = [pltpu.async_copy(
                           tab.at[idxv.at[g * CPG + t]],
                           rows[par].at[pl.ds(t * CH, CH), :],
                           sem_g)
                       for t in range(CPG)]
                for c in cps:
                    c.wait()
                pltpu.async_copy(rows[par], dst, sems[par])
            return carry
        lax.fori_loop(0, NG // 2, grp2, 0)
        for par in range(2):
            g = NG - 2 + par
            dst = out.at[pl.ds(base + g * GR, GR), pl.ds(f * D, D)]
            pltpu.make_async_copy(rows[par], dst, sems[par]).wait()

    lanes = lax.iota(jnp.int32, 16)
    for f, (tab_v, idxv) in enumerate(((cat_v, idx_c), (int_v, idx_x))):
        def grp2l(g2, carry, tab_v=tab_v, idxv=idxv, f=f + 2):
            for par in range(2):
                g = g2 * 2 + par
                dst = out.at[pl.ds(base + g * GR, GR), pl.ds(f * D, D)]

                @pl.when(g2 > 0)
                def _(par=par, dst=dst):
                    pltpu.make_async_copy(rows[par], dst, sems[par]).wait()

                def step(s, c2, tab_v=tab_v, idxv=idxv, par=par, g=g):
                    rowi = idxv[pl.ds(g * GR + s * 16, 16)]
                    ri = s * 16 + lanes
                    for col in range(D):
                        ci = jnp.full((16,), col, jnp.int32)
                        vals = plsc.load_gather(tab_v, [rowi, ci])
                        plsc.store_scatter(rows[par], [ri, ci], vals)
                    return c2
                lax.fori_loop(0, GR // 16, step, 0)
                pltpu.async_copy(rows[par], dst, sems[par])
            return carry
        lax.fori_loop(0, NG // 2, grp2l, 0)
        for par in range(2):
            g = NG - 2 + par
            dst = out.at[pl.ds(base + g * GR, GR), pl.ds((f + 2) * D, D)]
            pltpu.make_async_copy(rows[par], dst, sems[par]).wait()


def _sc_gather(t_u, t_i, t_c, t_x, ix_u, ix_i, ix_c, ix_x):
    mesh = plsc.VectorSubcoreMesh(core_axis_name="c", subcore_axis_name="s")
    fn = functools.partial(
        pl.kernel,
        mesh=mesh,
        out_type=jax.ShapeDtypeStruct((N, 4 * D), jnp.float32),
        scratch_types=[
            pltpu.VMEM((K, CH), jnp.int32),
            pltpu.VMEM((K, CH), jnp.int32),
            pltpu.VMEM((PW,), jnp.int32),
            pltpu.VMEM((PW,), jnp.int32),
            pltpu.VMEM((V_CAT1, D), jnp.float32),
            pltpu.VMEM((V_INT, D), jnp.float32),
            pltpu.VMEM((GR, D), jnp.float32),
            pltpu.VMEM((GR, D), jnp.float32),
            pltpu.SemaphoreType.DMA,
            pltpu.SemaphoreType.DMA,
            pltpu.SemaphoreType.DMA,
        ],
        compiler_params=pltpu.CompilerParams(
            use_tc_tiling_on_sc=False, needs_layout_passes=False),
    )(_sc_body)
    return fn(t_u, t_i, t_c, t_x, ix_u, ix_i, ix_c, ix_x)


TBLK = 8192


def _tr_body(t_ref, o_ref):
    xt = t_ref[...].T
    q = TBLK // 4
    o_ref[...] = jnp.concatenate(
        [xt[0:q], xt[q:2 * q], xt[2 * q:3 * q], xt[3 * q:4 * q]], axis=1)


def _to_rowmajor(tabT):
    V = tabT.shape[1]
    nb = (V + TBLK - 1) // TBLK
    out = pl.pallas_call(
        _tr_body,
        grid=(nb,),
        in_specs=[pl.BlockSpec((D, TBLK), lambda i: (0, i))],
        out_specs=pl.BlockSpec((TBLK // 4, 128), lambda i: (i, 0)),
        out_shape=jax.ShapeDtypeStruct((nb * TBLK // 4, 128), jnp.float32),
    )(tabT)
    return out.reshape(nb * TBLK, D)


BLK = 4096


def _mm_body(e_ref, w_ref, b_ref, o_ref):
    o_ref[...] = (
        jnp.dot(e_ref[...], w_ref[...], preferred_element_type=jnp.float32)
        + b_ref[...]
    )


def _project(e, W, b2):
    return pl.pallas_call(
        _mm_body,
        grid=(N // BLK,),
        in_specs=[
            pl.BlockSpec((BLK, 4 * D), lambda i: (i, 0)),
            pl.BlockSpec((4 * D, 128), lambda i: (0, 0)),
            pl.BlockSpec((1, 128), lambda i: (0, 0)),
        ],
        out_specs=pl.BlockSpec((BLK, 128), lambda i: (i, 0)),
        out_shape=jax.ShapeDtypeStruct((N, 128), jnp.float32),
    )(e, W, b2)


def kernel(user, item, category, interaction, emb_user, emb_item,
           emb_category, emb_interaction, W, b):
    def remap(v):
        v = v.astype(jnp.int32)
        u = v % TBLK
        return (v // TBLK) * TBLK + 4 * (u % (TBLK // 4)) + u // (TBLK // 4)

    ix_u = remap(user).T.reshape(NW, K, CH)
    ix_i = remap(item).T.reshape(NW, K, CH)
    ix_c = category.T.reshape(NW, K * CH).astype(jnp.int32)
    ix_x = interaction.T.reshape(NW, K * CH).astype(jnp.int32)
    tab_u = _to_rowmajor(emb_user.T)
    tab_i = _to_rowmajor(emb_item.T)
    e = _sc_gather(tab_u, tab_i, emb_category, emb_interaction,
                   ix_u, ix_i, ix_c, ix_x)
    out = _project(e, W, b.reshape(1, 128))
    return jnp.transpose(out.reshape(L, B, 128), (1, 0, 2))


# speedup vs baseline: 10.4477x; 1.2050x over previous
import functools

import jax
import jax.numpy as jnp
from jax import lax
from jax.experimental import pallas as pl
from jax.experimental.pallas import tpu as pltpu
from jax.experimental.pallas import tpu_sc as plsc

B, L, D = 4096, 50, 32
N = B * L
NC, NS = 2, 16
NW = NC * NS
PW = N // NW
CH = 128
CPG = 5
GR = CPG * CH
NG = PW // GR
K = PW // CH
V_CAT1 = 1001
V_INT = 3
TBLK = 8192


def _tr_body(t_ref, o_ref):
    xt = t_ref[...].T
    q = TBLK // 4
    o_ref[...] = jnp.concatenate(
        [xt[0:q], xt[q:2 * q], xt[2 * q:3 * q], xt[3 * q:4 * q]], axis=1)


def _to_rowmajor(tabT):
    V = tabT.shape[1]
    nb = (V + TBLK - 1) // TBLK
    out = pl.pallas_call(
        _tr_body,
        grid=(nb,),
        in_specs=[pl.BlockSpec((D, TBLK), lambda i: (0, i))],
        out_specs=pl.BlockSpec((TBLK // 4, 128), lambda i: (i, 0)),
        out_shape=jax.ShapeDtypeStruct((nb * TBLK // 4, 128), jnp.float32),
    )(tabT)
    return out.reshape(nb * TBLK, D)


def _stream_body(tab, ix, out_e, idx_v, rows0, rows1, sg0, sg1, sw0, sw1):
    wid = lax.axis_index("s") * NC + lax.axis_index("c")
    base = wid * PW
    pltpu.sync_copy(ix.at[wid], idx_v)
    rows = (rows0, rows1)
    sg = (sg0, sg1)
    sw = (sw0, sw1)

    def issue(g, par):
        for t in range(CPG):
            pltpu.async_copy(tab.at[idx_v.at[g * CPG + t]],
                             rows[par].at[pl.ds(t * CH, CH), :], sg[par])

    def drain(g, par):
        for t in range(CPG):
            pltpu.make_async_copy(tab.at[idx_v.at[g * CPG + t]],
                                  rows[par].at[pl.ds(t * CH, CH), :],
                                  sg[par]).wait()

    def wb(g, par):
        pltpu.async_copy(rows[par], out_e.at[pl.ds(base + g * GR, GR), :],
                         sw[par])

    def wb_wait(g, par):
        pltpu.make_async_copy(rows[par], out_e.at[pl.ds(base + g * GR, GR), :],
                              sw[par]).wait()

    issue(0, 0)

    def loop(g2, carry):
        for par in range(2):
            g = 2 * g2 + par
            if par == 0:
                @pl.when(g2 > 0)
                def _(g=g):
                    wb_wait(g - 1, 1)
            else:
                wb_wait(g - 1, 0)
            issue(g + 1, 1 - par)
            drain(g, par)
            wb(g, par)
        return carry

    lax.fori_loop(0, (NG - 2) // 2, loop, 0)
    wb_wait(NG - 3, 1)
    issue(NG - 1, 1)
    drain(NG - 2, 0)
    wb(NG - 2, 0)
    drain(NG - 1, 1)
    wb(NG - 1, 1)
    wb_wait(NG - 2, 0)
    wb_wait(NG - 1, 1)


def _stream_field(tab, ix):
    mesh = plsc.VectorSubcoreMesh(core_axis_name="c", subcore_axis_name="s")
    fn = functools.partial(
        pl.kernel,
        mesh=mesh,
        out_type=jax.ShapeDtypeStruct((N, D), jnp.float32),
        scratch_types=[
            pltpu.VMEM((K, CH), jnp.int32),
            pltpu.VMEM((GR, D), jnp.float32),
            pltpu.VMEM((GR, D), jnp.float32),
            pltpu.SemaphoreType.DMA,
            pltpu.SemaphoreType.DMA,
            pltpu.SemaphoreType.DMA,
            pltpu.SemaphoreType.DMA,
        ],
        compiler_params=pltpu.CompilerParams(
            use_tc_tiling_on_sc=False, needs_layout_passes=False),
    )(_stream_body)
    return fn(tab, ix)


def _ci_body(t_c, t_x, ix_c, ix_x, e_c, e_x,
             idx_c, idx_x, cat_v, int_v, rows0, rows1, sw0, sw1):
    wid = lax.axis_index("s") * NC + lax.axis_index("c")
    base = wid * PW
    pltpu.sync_copy(ix_c.at[wid], idx_c)
    pltpu.sync_copy(ix_x.at[wid], idx_x)
    pltpu.sync_copy(t_c, cat_v)
    pltpu.sync_copy(t_x, int_v)
    rows = (rows0, rows1)
    sems = (sw0, sw1)
    lanes = lax.iota(jnp.int32, 16)
    for tab_v, idxv, out_e in ((cat_v, idx_c, e_c), (int_v, idx_x, e_x)):
        def grp2(g2, carry, tab_v=tab_v, idxv=idxv, out_e=out_e):
            for par in range(2):
                g = g2 * 2 + par
                dst = out_e.at[pl.ds(base + g * GR, GR), :]

                @pl.when(g2 > 0)
                def _(par=par, dst=dst):
                    pltpu.make_async_copy(rows[par], dst, sems[par]).wait()

                def step(s, c2, tab_v=tab_v, idxv=idxv, par=par, g=g):
                    rowi = idxv[pl.ds(g * GR + s * 16, 16)]
                    ri = s * 16 + lanes
                    for col in range(D):
                        ci = jnp.full((16,), col, jnp.int32)
                        vals = plsc.load_gather(tab_v, [rowi, ci])
                        plsc.store_scatter(rows[par], [ri, ci], vals)
                    return c2
                lax.fori_loop(0, GR // 16, step, 0)
                pltpu.async_copy(rows[par], dst, sems[par])
            return carry
        lax.fori_loop(0, NG // 2, grp2, 0)
        for par in range(2):
            g = NG - 2 + par
            dst = out_e.at[pl.ds(base + g * GR, GR), :]
            pltpu.make_async_copy(rows[par], dst, sems[par]).wait()


def _ci_gather(t_c, t_x, ix_c, ix_x):
    mesh = plsc.VectorSubcoreMesh(core_axis_name="c", subcore_axis_name="s")
    fn = functools.partial(
        pl.kernel,
        mesh=mesh,
        out_type=(jax.ShapeDtypeStruct((N, D), jnp.float32),
                  jax.ShapeDtypeStruct((N, D), jnp.float32)),
        scratch_types=[
            pltpu.VMEM((PW,), jnp.int32),
            pltpu.VMEM((PW,), jnp.int32),
            pltpu.VMEM((V_CAT1, D), jnp.float32),
            pltpu.VMEM((V_INT, D), jnp.float32),
            pltpu.VMEM((GR, D), jnp.float32),
            pltpu.VMEM((GR, D), jnp.float32),
            pltpu.SemaphoreType.DMA,
            pltpu.SemaphoreType.DMA,
        ],
        compiler_params=pltpu.CompilerParams(
            use_tc_tiling_on_sc=False, needs_layout_passes=False),
    )(_ci_body)
    return fn(t_c, t_x, ix_c, ix_x)


BLK4 = 1024


def _mm_body(eu_ref, ei_ref, ec_ref, ex_ref, w_ref, b_ref, o_ref):
    acc = jnp.dot(eu_ref[...], w_ref[0], preferred_element_type=jnp.float32)
    acc = acc + jnp.dot(ei_ref[...], w_ref[1],
                        preferred_element_type=jnp.float32)
    acc = acc + jnp.dot(ec_ref[...], w_ref[2],
                        preferred_element_type=jnp.float32)
    acc = acc + jnp.dot(ex_ref[...], w_ref[3],
                        preferred_element_type=jnp.float32)
    o_ref[...] = acc + b_ref[...]


def _project(eu, ei, ec, ex, W4, b4):
    espec = pl.BlockSpec((BLK4, 128), lambda i: (i, 0))
    return pl.pallas_call(
        _mm_body,
        grid=(N // 4 // BLK4,),
        in_specs=[
            espec, espec, espec, espec,
            pl.BlockSpec((4, 128, 512), lambda i: (0, 0, 0)),
            pl.BlockSpec((1, 512), lambda i: (0, 0)),
        ],
        out_specs=pl.BlockSpec((BLK4, 512), lambda i: (i, 0)),
        out_shape=jax.ShapeDtypeStruct((N // 4, 512), jnp.float32),
    )(eu, ei, ec, ex, W4, b4)


def _kron4(Wf):
    return (jnp.eye(4, dtype=jnp.float32)[:, None, :, None]
            * Wf[None, :, None, :]).reshape(4 * D, 4 * 128)


def kernel(user, item, category, interaction, emb_user, emb_item,
           emb_category, emb_interaction, W, b):
    def remap(v):
        v = v.astype(jnp.int32)
        u = v % TBLK
        return (v // TBLK) * TBLK + 4 * (u % (TBLK // 4)) + u // (TBLK // 4)

    ix_u = remap(user).T.reshape(NW, K, CH)
    ix_i = remap(item).T.reshape(NW, K, CH)
    ix_c = category.T.reshape(NW, K * CH).astype(jnp.int32)
    ix_x = interaction.T.reshape(NW, K * CH).astype(jnp.int32)
    tab_u = _to_rowmajor(emb_user.T)
    tab_i = _to_rowmajor(emb_item.T)
    e_u = _stream_field(tab_u, ix_u)
    e_i = _stream_field(tab_i, ix_i)
    e_c, e_x = _ci_gather(emb_category, emb_interaction, ix_c, ix_x)
    W4 = jnp.stack([_kron4(W[f * D:(f + 1) * D, :]) for f in range(4)])
    b4 = jnp.tile(b.reshape(1, 128), (1, 4))
    pk = lambda a: a.reshape(N // 4, 128)
    out = _project(pk(e_u), pk(e_i), pk(e_c), pk(e_x), W4, b4)
    return jnp.transpose(out.reshape(L, B, 128), (1, 0, 2))


# speedup vs baseline: 13.8786x; 1.3284x over previous
import functools

import jax
import jax.numpy as jnp
from jax import lax
from jax.experimental import pallas as pl
from jax.experimental.pallas import tpu as pltpu
from jax.experimental.pallas import tpu_sc as plsc

B, L, D = 4096, 50, 32
N = B * L
N4 = N // 4
NC, NS = 2, 16
NW = NC * NS
PW = N // NW
CH = 128
CPG = 5
GR = CPG * CH
NG = PW // GR
K = PW // CH
V_CAT1 = 1001
CPAD = D + 1
TBLK = 8192


def _tr_body(t_ref, eye_ref, o_ref):
    x = t_ref[...]
    xt = lax.dot_general(x, eye_ref[...], (((0,), (0,)), ((), ())),
                         preferred_element_type=jnp.float32)
    q = TBLK // 4
    o_ref[...] = jnp.concatenate(
        [xt[0:q], xt[q:2 * q], xt[2 * q:3 * q], xt[3 * q:4 * q]], axis=1)


def _to_rowmajor(tabT, eye):
    V = tabT.shape[1]
    nb = (V + TBLK - 1) // TBLK
    out = pl.pallas_call(
        _tr_body,
        grid=(nb,),
        in_specs=[pl.BlockSpec((D, TBLK), lambda i: (0, i)),
                  pl.BlockSpec((D, D), lambda i: (0, 0))],
        out_specs=pl.BlockSpec((TBLK // 4, 128), lambda i: (i, 0)),
        out_shape=jax.ShapeDtypeStruct((nb * TBLK // 4, 128), jnp.float32),
        compiler_params=pltpu.CompilerParams(
            fuse_transposed_lhs_in_matmul=True),
    )(tabT, eye)
    return out.reshape(nb * TBLK, D)


def _stream_body(tab, ix, out_e, idx_v, rows0, rows1, sg0, sg1, sw0, sw1):
    wid = lax.axis_index("s") * NC + lax.axis_index("c")
    jcol = (wid // 8) * D
    r0 = (wid % 8) * PW
    pltpu.sync_copy(ix.at[wid], idx_v)
    rows = (rows0, rows1)
    sg = (sg0, sg1)
    sw = (sw0, sw1)

    def issue(g, par):
        for t in range(CPG):
            pltpu.async_copy(tab.at[idx_v.at[g * CPG + t]],
                             rows[par].at[pl.ds(t * CH, CH), :], sg[par])

    def drain(g, par):
        for t in range(CPG):
            pltpu.make_async_copy(tab.at[idx_v.at[g * CPG + t]],
                                  rows[par].at[pl.ds(t * CH, CH), :],
                                  sg[par]).wait()

    def wb(g, par):
        pltpu.async_copy(
            rows[par],
            out_e.at[pl.ds(r0 + g * GR, GR), pl.ds(jcol, D)], sw[par])

    def wb_wait(g, par):
        pltpu.make_async_copy(
            rows[par],
            out_e.at[pl.ds(r0 + g * GR, GR), pl.ds(jcol, D)], sw[par]).wait()

    issue(0, 0)

    def loop(g2, carry):
        for par in range(2):
            g = 2 * g2 + par
            if par == 0:
                @pl.when(g2 > 0)
                def _(g=g):
                    wb_wait(g - 1, 1)
            else:
                wb_wait(g - 1, 0)
            issue(g + 1, 1 - par)
            drain(g, par)
            wb(g, par)
        return carry

    lax.fori_loop(0, (NG - 2) // 2, loop, 0)
    wb_wait(NG - 3, 1)
    issue(NG - 1, 1)
    drain(NG - 2, 0)
    wb(NG - 2, 0)
    drain(NG - 1, 1)
    wb(NG - 1, 1)
    wb_wait(NG - 2, 0)
    wb_wait(NG - 1, 1)


def _stream_field(tab, ix):
    mesh = plsc.VectorSubcoreMesh(core_axis_name="c", subcore_axis_name="s")
    fn = functools.partial(
        pl.kernel,
        mesh=mesh,
        out_type=jax.ShapeDtypeStruct((N4, 128), jnp.float32),
        scratch_types=[
            pltpu.VMEM((K, CH), jnp.int32),
            pltpu.VMEM((GR, D), jnp.float32),
            pltpu.VMEM((GR, D), jnp.float32),
            pltpu.SemaphoreType.DMA,
            pltpu.SemaphoreType.DMA,
            pltpu.SemaphoreType.DMA,
            pltpu.SemaphoreType.DMA,
        ],
        compiler_params=pltpu.CompilerParams(
            use_tc_tiling_on_sc=False, needs_layout_passes=False),
    )(_stream_body)
    return fn(tab, ix)


def _cat_body(t_c, ix_c, e_c, idx_c, cat_v, rows0, rows1, sw0, sw1):
    wid = lax.axis_index("s") * NC + lax.axis_index("c")
    jcol = (wid // 8) * D
    r0 = (wid % 8) * PW
    pltpu.sync_copy(ix_c.at[wid], idx_c)
    pltpu.sync_copy(t_c, cat_v.at[:, pl.ds(0, D)])
    rows = (rows0, rows1)
    sems = (sw0, sw1)
    lanes = lax.iota(jnp.int32, 16)

    def grp2(g2, carry):
        for par in range(2):
            g = g2 * 2 + par
            dst = e_c.at[pl.ds(r0 + g * GR, GR), pl.ds(jcol, D)]

            @pl.when(g2 > 0)
            def _(par=par, dst=dst):
                pltpu.make_async_copy(rows[par].at[:, pl.ds(0, D)], dst,
                                      sems[par]).wait()

            def step(s, c2, par=par, g=g):
                rowi = idx_c[pl.ds(g * GR + s * 16, 16)]
                ri = s * 16 + lanes
                for col in range(D):
                    ci = jnp.full((16,), col, jnp.int32)
                    vals = plsc.load_gather(cat_v, [rowi, ci])
                    plsc.store_scatter(rows[par], [ri, ci], vals)
                return c2
            lax.fori_loop(0, GR // 16, step, 0)
            pltpu.async_copy(rows[par].at[:, pl.ds(0, D)], dst, sems[par])
        return carry

    lax.fori_loop(0, NG // 2, grp2, 0)
    for par in range(2):
        g = NG - 2 + par
        dst = e_c.at[pl.ds(r0 + g * GR, GR), pl.ds(jcol, D)]
        pltpu.make_async_copy(rows[par].at[:, pl.ds(0, D)], dst,
                              sems[par]).wait()


def _cat_gather(t_c, ix_c):
    mesh = plsc.VectorSubcoreMesh(core_axis_name="c", subcore_axis_name="s")
    fn = functools.partial(
        pl.kernel,
        mesh=mesh,
        out_type=jax.ShapeDtypeStruct((N4, 128), jnp.float32),
        scratch_types=[
            pltpu.VMEM((PW,), jnp.int32),
            pltpu.VMEM((V_CAT1, CPAD), jnp.float32),
            pltpu.VMEM((GR, CPAD), jnp.float32),
            pltpu.VMEM((GR, CPAD), jnp.float32),
            pltpu.SemaphoreType.DMA,
            pltpu.SemaphoreType.DMA,
        ],
        compiler_params=pltpu.CompilerParams(
            use_tc_tiling_on_sc=False, needs_layout_passes=False),
    )(_cat_body)
    return fn(t_c, ix_c)


BLK4 = 1024


def _mm_body(eu_ref, ei_ref, ec_ref, xi_ref, w_ref, tx_ref, wx_ref, b_ref,
             o_ref):
    acc = jnp.dot(eu_ref[...], w_ref[0], preferred_element_type=jnp.float32)
    acc = acc + jnp.dot(ei_ref[...], w_ref[1],
                        preferred_element_type=jnp.float32)
    acc = acc + jnp.dot(ec_ref[...], w_ref[2],
                        preferred_element_type=jnp.float32)
    pint = jnp.dot(tx_ref[...], wx_ref[...],
                   preferred_element_type=jnp.float32)
    xb = xi_ref[...]
    for j in range(4):
        xj = xb[:, j:j + 1]
        contrib = jnp.where(
            xj == 0, pint[0:1, :],
            jnp.where(xj == 1, pint[1:2, :], pint[2:3, :]))
        o_ref[j] = acc[:, j * 128:(j + 1) * 128] + contrib + b_ref[...]


def _project(eu, ei, ec, xi, W3, tx, wx, b1):
    espec = pl.BlockSpec((BLK4, 128), lambda i: (i, 0))
    return pl.pallas_call(
        _mm_body,
        grid=(N4 // BLK4,),
        in_specs=[
            espec, espec, espec,
            pl.BlockSpec((BLK4, 4), lambda i: (i, 0)),
            pl.BlockSpec((3, 128, 512), lambda i: (0, 0, 0)),
            pl.BlockSpec((3, D), lambda i: (0, 0)),
            pl.BlockSpec((D, 128), lambda i: (0, 0)),
            pl.BlockSpec((1, 128), lambda i: (0, 0)),
        ],
        out_specs=pl.BlockSpec((4, BLK4, 128), lambda i: (0, i, 0)),
        out_shape=jax.ShapeDtypeStruct((4, N4, 128), jnp.float32),
    )(eu, ei, ec, xi, W3, tx, wx, b1)


def _kron4(Wf):
    return (jnp.eye(4, dtype=jnp.float32)[:, None, :, None]
            * Wf[None, :, None, :]).reshape(4 * D, 4 * 128)


def kernel(user, item, category, interaction, emb_user, emb_item,
           emb_category, emb_interaction, W, b):
    def remap(v):
        v = v.astype(jnp.int32)
        u = v % TBLK
        return (v // TBLK) * TBLK + 4 * (u % (TBLK // 4)) + u // (TBLK // 4)

    ix_u = remap(user).T.reshape(NW, K, CH)
    ix_i = remap(item).T.reshape(NW, K, CH)
    ix_c = category.T.reshape(NW, K * CH).astype(jnp.int32)
    xi = interaction.T.reshape(4, N4).T.astype(jnp.int32)
    eye = jnp.eye(D, dtype=jnp.float32)
    tab_u = _to_rowmajor(emb_user.T, eye)
    tab_i = _to_rowmajor(emb_item.T, eye)
    e_u = _stream_field(tab_u, ix_u)
    e_i = _stream_field(tab_i, ix_i)
    e_c = _cat_gather(emb_category, ix_c)
    W3 = jnp.stack([_kron4(W[f * D:(f + 1) * D, :]) for f in range(3)])
    out = _project(e_u, e_i, e_c, xi, W3, emb_interaction,
                   W[3 * D:4 * D, :], b.reshape(1, 128))
    return jnp.transpose(out.reshape(L, B, 128), (1, 0, 2))


# speedup vs baseline: 17.3098x; 1.2472x over previous
import functools

import jax
import jax.numpy as jnp
from jax import lax
from jax.experimental import pallas as pl
from jax.experimental.pallas import tpu as pltpu
from jax.experimental.pallas import tpu_sc as plsc

B, L, D = 4096, 50, 32
N = B * L
N4 = N // 4
NC, NS = 2, 16
NW = NC * NS
PW = N // NW
CH = 128
CPG = 5
GR = CPG * CH
NG = PW // GR
K = PW // CH
V_CAT1 = 1001
CPAD = D + 1
TBLK = 8192


def _tr_body(t_ref, eye_ref, o_ref):
    x = t_ref[...]
    xt = lax.dot_general(x, eye_ref[...], (((0,), (0,)), ((), ())),
                         preferred_element_type=jnp.float32)
    q = TBLK // 4
    o_ref[...] = jnp.concatenate(
        [xt[0:q], xt[q:2 * q], xt[2 * q:3 * q], xt[3 * q:4 * q]], axis=1)


def _to_rowmajor(tabT, eye):
    V = tabT.shape[1]
    nb = (V + TBLK - 1) // TBLK
    out = pl.pallas_call(
        _tr_body,
        grid=(nb,),
        in_specs=[pl.BlockSpec((D, TBLK), lambda i: (0, i)),
                  pl.BlockSpec((D, D), lambda i: (0, 0))],
        out_specs=pl.BlockSpec((TBLK // 4, 128), lambda i: (i, 0)),
        out_shape=jax.ShapeDtypeStruct((nb * TBLK // 4, 128), jnp.float32),
        compiler_params=pltpu.CompilerParams(
            fuse_transposed_lhs_in_matmul=True),
    )(tabT, eye)
    return out.reshape(nb * TBLK, D)


def _stream_body(tab, ix, out_e, idx_v, rows0, rows1, sg0, sg1, sw0, sw1):
    wid = lax.axis_index("s") * NC + lax.axis_index("c")
    jcol = (wid // 8) * D
    r0 = (wid % 8) * PW
    pltpu.sync_copy(ix.at[wid], idx_v)
    rows = (rows0, rows1)
    sg = (sg0, sg1)
    sw = (sw0, sw1)

    def issue(g, par):
        for t in range(CPG):
            pltpu.async_copy(tab.at[idx_v.at[g * CPG + t]],
                             rows[par].at[pl.ds(t * CH, CH), :], sg[par])

    def drain(g, par):
        for t in range(CPG):
            pltpu.make_async_copy(tab.at[idx_v.at[g * CPG + t]],
                                  rows[par].at[pl.ds(t * CH, CH), :],
                                  sg[par]).wait()

    def wb(g, par):
        pltpu.async_copy(
            rows[par],
            out_e.at[pl.ds(r0 + g * GR, GR), pl.ds(jcol, D)], sw[par])

    def wb_wait(g, par):
        pltpu.make_async_copy(
            rows[par],
            out_e.at[pl.ds(r0 + g * GR, GR), pl.ds(jcol, D)], sw[par]).wait()

    issue(0, 0)

    def loop(g2, carry):
        for par in range(2):
            g = 2 * g2 + par
            if par == 0:
                @pl.when(g2 > 0)
                def _(g=g):
                    wb_wait(g - 1, 1)
            else:
                wb_wait(g - 1, 0)
            issue(g + 1, 1 - par)
            drain(g, par)
            wb(g, par)
        return carry

    lax.fori_loop(0, (NG - 2) // 2, loop, 0)
    wb_wait(NG - 3, 1)
    issue(NG - 1, 1)
    drain(NG - 2, 0)
    wb(NG - 2, 0)
    drain(NG - 1, 1)
    wb(NG - 1, 1)
    wb_wait(NG - 2, 0)
    wb_wait(NG - 1, 1)


def _stream_field(tab, ix):
    mesh = plsc.VectorSubcoreMesh(core_axis_name="c", subcore_axis_name="s")
    fn = functools.partial(
        pl.kernel,
        mesh=mesh,
        out_type=jax.ShapeDtypeStruct((N4, 128), jnp.float32),
        scratch_types=[
            pltpu.VMEM((K, CH), jnp.int32),
            pltpu.VMEM((GR, D), jnp.float32),
            pltpu.VMEM((GR, D), jnp.float32),
            pltpu.SemaphoreType.DMA,
            pltpu.SemaphoreType.DMA,
            pltpu.SemaphoreType.DMA,
            pltpu.SemaphoreType.DMA,
        ],
        compiler_params=pltpu.CompilerParams(
            use_tc_tiling_on_sc=False, needs_layout_passes=False),
    )(_stream_body)
    return fn(tab, ix)


def _cat_body(t_c, ix_c, e_c, idx_v, cat_v, rows0, rows1,
              sg0, sg1, sw0, sw1):
    wid = lax.axis_index("s") * NC + lax.axis_index("c")
    jcol = (wid // 8) * D
    r0 = (wid % 8) * PW
    pltpu.sync_copy(ix_c.at[wid], idx_v)

    @pl.when(lax.axis_index("s") == 0)
    def _():
        pltpu.sync_copy(t_c, cat_v)
    plsc.subcore_barrier()
    rows = (rows0, rows1)
    sg = (sg0, sg1)
    sw = (sw0, sw1)

    def issue(g, par):
        for t in range(CPG):
            pltpu.async_copy(cat_v.at[idx_v.at[g * CPG + t]],
                             rows[par].at[pl.ds(t * CH, CH), :], sg[par])

    def drain(g, par):
        for t in range(CPG):
            pltpu.make_async_copy(cat_v.at[idx_v.at[g * CPG + t]],
                                  rows[par].at[pl.ds(t * CH, CH), :],
                                  sg[par]).wait()

    def wb(g, par):
        pltpu.async_copy(
            rows[par],
            e_c.at[pl.ds(r0 + g * GR, GR), pl.ds(jcol, D)], sw[par])

    def wb_wait(g, par):
        pltpu.make_async_copy(
            rows[par],
            e_c.at[pl.ds(r0 + g * GR, GR), pl.ds(jcol, D)], sw[par]).wait()

    issue(0, 0)

    def loop(g2, carry):
        for par in range(2):
            g = 2 * g2 + par
            if par == 0:
                @pl.when(g2 > 0)
                def _(g=g):
                    wb_wait(g - 1, 1)
            else:
                wb_wait(g - 1, 0)
            issue(g + 1, 1 - par)
            drain(g, par)
            wb(g, par)
        return carry

    lax.fori_loop(0, (NG - 2) // 2, loop, 0)
    wb_wait(NG - 3, 1)
    issue(NG - 1, 1)
    drain(NG - 2, 0)
    wb(NG - 2, 0)
    drain(NG - 1, 1)
    wb(NG - 1, 1)
    wb_wait(NG - 2, 0)
    wb_wait(NG - 1, 1)


def _cat_gather(t_c, ix_c):
    mesh = plsc.VectorSubcoreMesh(core_axis_name="c", subcore_axis_name="s")
    fn = functools.partial(
        pl.kernel,
        mesh=mesh,
        out_type=jax.ShapeDtypeStruct((N4, 128), jnp.float32),
        scratch_types=[
            pltpu.VMEM((K, CH), jnp.int32),
            pltpu.VMEM_SHARED((V_CAT1, D), jnp.float32),
            pltpu.VMEM((GR, D), jnp.float32),
            pltpu.VMEM((GR, D), jnp.float32),
            pltpu.SemaphoreType.DMA,
            pltpu.SemaphoreType.DMA,
            pltpu.SemaphoreType.DMA,
            pltpu.SemaphoreType.DMA,
        ],
        compiler_params=pltpu.CompilerParams(
            use_tc_tiling_on_sc=False, needs_layout_passes=False),
    )(_cat_body)
    return fn(t_c, ix_c)


BLK4 = 1024


def _mm_body(eu_ref, ei_ref, ec_ref, xi_ref, w_ref, tx_ref, wx_ref, b_ref,
             o_ref):
    acc = jnp.dot(eu_ref[...], w_ref[0], preferred_element_type=jnp.float32)
    acc = acc + jnp.dot(ei_ref[...], w_ref[1],
                        preferred_element_type=jnp.float32)
    acc = acc + jnp.dot(ec_ref[...], w_ref[2],
                        preferred_element_type=jnp.float32)
    pint = jnp.dot(tx_ref[...], wx_ref[...],
                   preferred_element_type=jnp.float32)
    xb = xi_ref[...]
    for j in range(4):
        xj = xb[:, j:j + 1]
        contrib = jnp.where(
            xj == 0, pint[0:1, :],
            jnp.where(xj == 1, pint[1:2, :], pint[2:3, :]))
        o_ref[j] = acc[:, j * 128:(j + 1) * 128] + contrib + b_ref[...]


def _project(eu, ei, ec, xi, W3, tx, wx, b1):
    espec = pl.BlockSpec((BLK4, 128), lambda i: (i, 0))
    return pl.pallas_call(
        _mm_body,
        grid=(N4 // BLK4,),
        in_specs=[
            espec, espec, espec,
            pl.BlockSpec((BLK4, 4), lambda i: (i, 0)),
            pl.BlockSpec((3, 128, 512), lambda i: (0, 0, 0)),
            pl.BlockSpec((3, D), lambda i: (0, 0)),
            pl.BlockSpec((D, 128), lambda i: (0, 0)),
            pl.BlockSpec((1, 128), lambda i: (0, 0)),
        ],
        out_specs=pl.BlockSpec((4, BLK4, 128), lambda i: (0, i, 0)),
        out_shape=jax.ShapeDtypeStruct((4, N4, 128), jnp.float32),
    )(eu, ei, ec, xi, W3, tx, wx, b1)


def _kron4(Wf):
    return (jnp.eye(4, dtype=jnp.float32)[:, None, :, None]
            * Wf[None, :, None, :]).reshape(4 * D, 4 * 128)


def kernel(user, item, category, interaction, emb_user, emb_item,
           emb_category, emb_interaction, W, b):
    def remap(v):
        v = v.astype(jnp.int32)
        u = v % TBLK
        return (v // TBLK) * TBLK + 4 * (u % (TBLK // 4)) + u // (TBLK // 4)

    ix_u = remap(user).T.reshape(NW, K, CH)
    ix_i = remap(item).T.reshape(NW, K, CH)
    ix_c = category.T.reshape(NW, K, CH).astype(jnp.int32)
    xi = interaction.T.reshape(4, N4).T.astype(jnp.int32)
    eye = jnp.eye(D, dtype=jnp.float32)
    e_c = _cat_gather(emb_category, ix_c)
    tab_i = _to_rowmajor(emb_item.T, eye)
    e_i = _stream_field(tab_i, ix_i)
    tab_u = _to_rowmajor(emb_user.T, eye)
    e_u = _stream_field(tab_u, ix_u)
    W3 = jnp.stack([_kron4(W[f * D:(f + 1) * D, :]) for f in range(3)])
    out = _project(e_u, e_i, e_c, xi, W3, emb_interaction,
                   W[3 * D:4 * D, :], b.reshape(1, 128))
    return jnp.transpose(out.reshape(L, B, 128), (1, 0, 2))


# speedup vs baseline: 23.2176x; 1.3413x over previous
import functools

import jax
import jax.numpy as jnp
from jax import lax
from jax.experimental import pallas as pl
from jax.experimental.pallas import tpu as pltpu
from jax.experimental.pallas import tpu_sc as plsc

B, L, D = 4096, 50, 32
N = B * L
N4 = N // 4
NC, NS = 2, 16
NW = NC * NS
PW = N // NW
CH = 128
CPG = 5
GR = CPG * CH
NG = PW // GR
K = PW // CH
V_CAT1 = 1001
CPAD = D + 1
TBLK = 8192


def _tr_body(t_ref, eye_ref, o_ref):
    x = t_ref[...]
    q = TBLK // 4
    x4 = jnp.concatenate(
        [x[:, 0:q], x[:, q:2 * q], x[:, 2 * q:3 * q], x[:, 3 * q:4 * q]],
        axis=0)
    o_ref[...] = lax.dot_general(x4, eye_ref[...], (((0,), (0,)), ((), ())),
                                 preferred_element_type=jnp.float32)


def _to_rowmajor(tabT, eye):
    V = tabT.shape[1]
    nb = (V + TBLK - 1) // TBLK
    out = pl.pallas_call(
        _tr_body,
        grid=(nb,),
        in_specs=[pl.BlockSpec((D, TBLK), lambda i: (0, i)),
                  pl.BlockSpec((128, 128), lambda i: (0, 0))],
        out_specs=pl.BlockSpec((TBLK // 4, 128), lambda i: (i, 0)),
        out_shape=jax.ShapeDtypeStruct((nb * TBLK // 4, 128), jnp.float32),
        compiler_params=pltpu.CompilerParams(
            fuse_transposed_lhs_in_matmul=True),
    )(tabT, eye)
    return out.reshape(nb * TBLK, D)


def _stream_body(tab, ix, out_e, idx_v, rows0, rows1, sg0, sg1, sw0, sw1):
    wid = lax.axis_index("s") * NC + lax.axis_index("c")
    jcol = (wid // 8) * D
    r0 = (wid % 8) * PW
    pltpu.sync_copy(ix.at[wid], idx_v)
    rows = (rows0, rows1)
    sg = (sg0, sg1)
    sw = (sw0, sw1)

    def issue(g, par):
        for t in range(CPG):
            pltpu.async_copy(tab.at[idx_v.at[g * CPG + t]],
                             rows[par].at[pl.ds(t * CH, CH), :], sg[par])

    def drain(g, par):
        for t in range(CPG):
            pltpu.make_async_copy(tab.at[idx_v.at[g * CPG + t]],
                                  rows[par].at[pl.ds(t * CH, CH), :],
                                  sg[par]).wait()

    def wb(g, par):
        pltpu.async_copy(
            rows[par],
            out_e.at[pl.ds(r0 + g * GR, GR), pl.ds(jcol, D)], sw[par])

    def wb_wait(g, par):
        pltpu.make_async_copy(
            rows[par],
            out_e.at[pl.ds(r0 + g * GR, GR), pl.ds(jcol, D)], sw[par]).wait()

    issue(0, 0)

    def loop(g2, carry):
        for par in range(2):
            g = 2 * g2 + par
            if par == 0:
                @pl.when(g2 > 0)
                def _(g=g):
                    wb_wait(g - 1, 1)
            else:
                wb_wait(g - 1, 0)
            issue(g + 1, 1 - par)
            drain(g, par)
            wb(g, par)
        return carry

    lax.fori_loop(0, (NG - 2) // 2, loop, 0)
    wb_wait(NG - 3, 1)
    issue(NG - 1, 1)
    drain(NG - 2, 0)
    wb(NG - 2, 0)
    drain(NG - 1, 1)
    wb(NG - 1, 1)
    wb_wait(NG - 2, 0)
    wb_wait(NG - 1, 1)


def _stream_field(tab, ix):
    mesh = plsc.VectorSubcoreMesh(core_axis_name="c", subcore_axis_name="s")
    fn = functools.partial(
        pl.kernel,
        mesh=mesh,
        out_type=jax.ShapeDtypeStruct((N4, 128), jnp.float32),
        scratch_types=[
            pltpu.VMEM((K, CH), jnp.int32),
            pltpu.VMEM((GR, D), jnp.float32),
            pltpu.VMEM((GR, D), jnp.float32),
            pltpu.SemaphoreType.DMA,
            pltpu.SemaphoreType.DMA,
            pltpu.SemaphoreType.DMA,
            pltpu.SemaphoreType.DMA,
        ],
        compiler_params=pltpu.CompilerParams(
            use_tc_tiling_on_sc=False, needs_layout_passes=False),
    )(_stream_body)
    return fn(tab, ix)


def _cat_body(t_c, ix_c, e_c, idx_v, cat_v, rows0, rows1,
              sg0, sg1, sw0, sw1):
    wid = lax.axis_index("s") * NC + lax.axis_index("c")
    jcol = (wid // 8) * D
    r0 = (wid % 8) * PW
    pltpu.sync_copy(ix_c.at[wid], idx_v)

    @pl.when(lax.axis_index("s") == 0)
    def _():
        pltpu.sync_copy(t_c, cat_v)
    plsc.subcore_barrier()
    rows = (rows0, rows1)
    sg = (sg0, sg1)
    sw = (sw0, sw1)

    def issue(g, par):
        for t in range(CPG):
            pltpu.async_copy(cat_v.at[idx_v.at[g * CPG + t]],
                             rows[par].at[pl.ds(t * CH, CH), :], sg[par])

    def drain(g, par):
        for t in range(CPG):
            pltpu.make_async_copy(cat_v.at[idx_v.at[g * CPG + t]],
                                  rows[par].at[pl.ds(t * CH, CH), :],
                                  sg[par]).wait()

    def wb(g, par):
        pltpu.async_copy(
            rows[par],
            e_c.at[pl.ds(r0 + g * GR, GR), pl.ds(jcol, D)], sw[par])

    def wb_wait(g, par):
        pltpu.make_async_copy(
            rows[par],
            e_c.at[pl.ds(r0 + g * GR, GR), pl.ds(jcol, D)], sw[par]).wait()

    issue(0, 0)

    def loop(g2, carry):
        for par in range(2):
            g = 2 * g2 + par
            if par == 0:
                @pl.when(g2 > 0)
                def _(g=g):
                    wb_wait(g - 1, 1)
            else:
                wb_wait(g - 1, 0)
            issue(g + 1, 1 - par)
            drain(g, par)
            wb(g, par)
        return carry

    lax.fori_loop(0, (NG - 2) // 2, loop, 0)
    wb_wait(NG - 3, 1)
    issue(NG - 1, 1)
    drain(NG - 2, 0)
    wb(NG - 2, 0)
    drain(NG - 1, 1)
    wb(NG - 1, 1)
    wb_wait(NG - 2, 0)
    wb_wait(NG - 1, 1)


def _cat_gather(t_c, ix_c):
    mesh = plsc.VectorSubcoreMesh(core_axis_name="c", subcore_axis_name="s")
    fn = functools.partial(
        pl.kernel,
        mesh=mesh,
        out_type=jax.ShapeDtypeStruct((N4, 128), jnp.float32),
        scratch_types=[
            pltpu.VMEM((K, CH), jnp.int32),
            pltpu.VMEM_SHARED((V_CAT1, D), jnp.float32),
            pltpu.VMEM((GR, D), jnp.float32),
            pltpu.VMEM((GR, D), jnp.float32),
            pltpu.SemaphoreType.DMA,
            pltpu.SemaphoreType.DMA,
            pltpu.SemaphoreType.DMA,
            pltpu.SemaphoreType.DMA,
        ],
        compiler_params=pltpu.CompilerParams(
            use_tc_tiling_on_sc=False, needs_layout_passes=False),
    )(_cat_body)
    return fn(t_c, ix_c)


BLK4 = 1024


def _mm_body(eu_ref, ei_ref, ec_ref, xi_ref, w_ref, tx_ref, wx_ref, b_ref,
             o_ref):
    acc = jnp.dot(eu_ref[...], w_ref[0], preferred_element_type=jnp.float32)
    acc = acc + jnp.dot(ei_ref[...], w_ref[1],
                        preferred_element_type=jnp.float32)
    acc = acc + jnp.dot(ec_ref[...], w_ref[2],
                        preferred_element_type=jnp.float32)
    pint = jnp.dot(tx_ref[...], wx_ref[...],
                   preferred_element_type=jnp.float32)
    xb = xi_ref[...]
    for j in range(4):
        xj = xb[:, j:j + 1]
        contrib = jnp.where(
            xj == 0, pint[0:1, :],
            jnp.where(xj == 1, pint[1:2, :], pint[2:3, :]))
        o_ref[j] = acc[:, j * 128:(j + 1) * 128] + contrib + b_ref[...]


def _project(eu, ei, ec, xi, W3, tx, wx, b1):
    espec = pl.BlockSpec((BLK4, 128), lambda i: (i, 0))
    return pl.pallas_call(
        _mm_body,
        grid=(N4 // BLK4,),
        in_specs=[
            espec, espec, espec,
            pl.BlockSpec((BLK4, 4), lambda i: (i, 0)),
            pl.BlockSpec((3, 128, 512), lambda i: (0, 0, 0)),
            pl.BlockSpec((3, D), lambda i: (0, 0)),
            pl.BlockSpec((D, 128), lambda i: (0, 0)),
            pl.BlockSpec((1, 128), lambda i: (0, 0)),
        ],
        out_specs=pl.BlockSpec((4, BLK4, 128), lambda i: (0, i, 0)),
        out_shape=jax.ShapeDtypeStruct((4, N4, 128), jnp.float32),
    )(eu, ei, ec, xi, W3, tx, wx, b1)


def _kron4(Wf):
    return (jnp.eye(4, dtype=jnp.float32)[:, None, :, None]
            * Wf[None, :, None, :]).reshape(4 * D, 4 * 128)


def kernel(user, item, category, interaction, emb_user, emb_item,
           emb_category, emb_interaction, W, b):
    def remap(v):
        v = v.astype(jnp.int32)
        u = v % TBLK
        return (v // TBLK) * TBLK + 4 * (u % (TBLK // 4)) + u // (TBLK // 4)

    ix_u = remap(user).T.reshape(NW, K, CH)
    ix_i = remap(item).T.reshape(NW, K, CH)
    ix_c = category.T.reshape(NW, K, CH).astype(jnp.int32)
    xi = interaction.T.reshape(4, N4).T.astype(jnp.int32)
    eye = jnp.eye(128, dtype=jnp.float32)
    e_c = _cat_gather(emb_category, ix_c)
    tab_i = _to_rowmajor(emb_item.T, eye)
    e_i = _stream_field(tab_i, ix_i)
    tab_u = _to_rowmajor(emb_user.T, eye)
    e_u = _stream_field(tab_u, ix_u)
    W3 = jnp.stack([_kron4(W[f * D:(f + 1) * D, :]) for f in range(3)])
    out = _project(e_u, e_i, e_c, xi, W3, emb_interaction,
                   W[3 * D:4 * D, :], b.reshape(1, 128))
    return jnp.transpose(out.reshape(L, B, 128), (1, 0, 2))


# speedup vs baseline: 24.9462x; 1.0745x over previous
import functools

import jax
import jax.numpy as jnp
from jax import lax
from jax.experimental import pallas as pl
from jax.experimental.pallas import tpu as pltpu
from jax.experimental.pallas import tpu_sc as plsc

B, L, D = 4096, 50, 32
N = B * L
N4 = N // 4
NC, NS = 2, 16
NW = NC * NS
PW = N // NW
CH = 128
CPG = 5
GR = CPG * CH
NG = PW // GR
K = PW // CH
V_CAT1 = 1001
CPAD = D + 1
TBLK = 16384


def _tr_body(t_ref, eye_ref, o_ref):
    x = t_ref[...]
    q = TBLK // 4
    x4 = jnp.concatenate(
        [x[:, 0:q], x[:, q:2 * q], x[:, 2 * q:3 * q], x[:, 3 * q:4 * q]],
        axis=0)
    o_ref[...] = lax.dot_general(x4, eye_ref[...], (((0,), (0,)), ((), ())),
                                 preferred_element_type=jnp.float32)


def _to_rowmajor(tabT, eye):
    V = tabT.shape[1]
    nb = (V + TBLK - 1) // TBLK
    out = pl.pallas_call(
        _tr_body,
        grid=(nb,),
        in_specs=[pl.BlockSpec((D, TBLK), lambda i: (0, i)),
                  pl.BlockSpec((128, 128), lambda i: (0, 0))],
        out_specs=pl.BlockSpec((TBLK // 4, 128), lambda i: (i, 0)),
        out_shape=jax.ShapeDtypeStruct((nb * TBLK // 4, 128), jnp.float32),
        compiler_params=pltpu.CompilerParams(
            fuse_transposed_lhs_in_matmul=True),
    )(tabT, eye)
    return out.reshape(nb * TBLK, D)


def _stream_body(tab, ix, out_e, idx_v, rows0, rows1, sg0, sg1, sw0, sw1):
    wid = lax.axis_index("s") * NC + lax.axis_index("c")
    jcol = (wid // 8) * D
    r0 = (wid % 8) * PW
    pltpu.sync_copy(ix.at[wid], idx_v)
    rows = (rows0, rows1)
    sg = (sg0, sg1)
    sw = (sw0, sw1)

    def issue(g, par):
        for t in range(CPG):
            pltpu.async_copy(tab.at[idx_v.at[g * CPG + t]],
                             rows[par].at[pl.ds(t * CH, CH), :], sg[par])

    def drain(g, par):
        for t in range(CPG):
            pltpu.make_async_copy(tab.at[idx_v.at[g * CPG + t]],
                                  rows[par].at[pl.ds(t * CH, CH), :],
                                  sg[par]).wait()

    def wb(g, par):
        pltpu.async_copy(
            rows[par],
            out_e.at[pl.ds(r0 + g * GR, GR), pl.ds(jcol, D)], sw[par])

    def wb_wait(g, par):
        pltpu.make_async_copy(
            rows[par],
            out_e.at[pl.ds(r0 + g * GR, GR), pl.ds(jcol, D)], sw[par]).wait()

    issue(0, 0)

    def loop(g2, carry):
        for par in range(2):
            g = 2 * g2 + par
            if par == 0:
                @pl.when(g2 > 0)
                def _(g=g):
                    wb_wait(g - 1, 1)
            else:
                wb_wait(g - 1, 0)
            issue(g + 1, 1 - par)
            drain(g, par)
            wb(g, par)
        return carry

    lax.fori_loop(0, (NG - 2) // 2, loop, 0)
    wb_wait(NG - 3, 1)
    issue(NG - 1, 1)
    drain(NG - 2, 0)
    wb(NG - 2, 0)
    drain(NG - 1, 1)
    wb(NG - 1, 1)
    wb_wait(NG - 2, 0)
    wb_wait(NG - 1, 1)


def _stream_field(tab, ix):
    mesh = plsc.VectorSubcoreMesh(core_axis_name="c", subcore_axis_name="s")
    fn = functools.partial(
        pl.kernel,
        mesh=mesh,
        out_type=jax.ShapeDtypeStruct((N4, 128), jnp.float32),
        scratch_types=[
            pltpu.VMEM((K, CH), jnp.int32),
            pltpu.VMEM((GR, D), jnp.float32),
            pltpu.VMEM((GR, D), jnp.float32),
            pltpu.SemaphoreType.DMA,
            pltpu.SemaphoreType.DMA,
            pltpu.SemaphoreType.DMA,
            pltpu.SemaphoreType.DMA,
        ],
        compiler_params=pltpu.CompilerParams(
            use_tc_tiling_on_sc=False, needs_layout_passes=False),
    )(_stream_body)
    return fn(tab, ix)


def _cat_body(t_c, t_x, ix_c, ix_x, e_c, e_x, idx_c, idx_x,
              cat_v, int_v, rows0, rows1, sg0, sg1, sw0, sw1):
    wid = lax.axis_index("s") * NC + lax.axis_index("c")
    jcol = (wid // 8) * D
    r0 = (wid % 8) * PW
    pltpu.sync_copy(ix_c.at[wid], idx_c)
    pltpu.sync_copy(ix_x.at[wid], idx_x)

    @pl.when(lax.axis_index("s") == 0)
    def _():
        pltpu.sync_copy(t_c, cat_v)
        pltpu.sync_copy(t_x, int_v)
    plsc.subcore_barrier()
    rows = (rows0, rows1)
    sg = (sg0, sg1)
    sw = (sw0, sw1)

    for tab_v, idx_v, out_e in ((cat_v, idx_c, e_c), (int_v, idx_x, e_x)):
        def issue(g, par, tab_v=tab_v, idx_v=idx_v):
            for t in range(CPG):
                pltpu.async_copy(tab_v.at[idx_v.at[g * CPG + t]],
                                 rows[par].at[pl.ds(t * CH, CH), :], sg[par])

        def drain(g, par, tab_v=tab_v, idx_v=idx_v):
            for t in range(CPG):
                pltpu.make_async_copy(tab_v.at[idx_v.at[g * CPG + t]],
                                      rows[par].at[pl.ds(t * CH, CH), :],
                                      sg[par]).wait()

        def wb(g, par, out_e=out_e):
            pltpu.async_copy(
                rows[par],
                out_e.at[pl.ds(r0 + g * GR, GR), pl.ds(jcol, D)], sw[par])

        def wb_wait(g, par, out_e=out_e):
            pltpu.make_async_copy(
                rows[par],
                out_e.at[pl.ds(r0 + g * GR, GR), pl.ds(jcol, D)],
                sw[par]).wait()

        issue(0, 0)

        def loop(g2, carry):
            for par in range(2):
                g = 2 * g2 + par
                if par == 0:
                    @pl.when(g2 > 0)
                    def _(g=g):
                        wb_wait(g - 1, 1)
                else:
                    wb_wait(g - 1, 0)
                issue(g + 1, 1 - par)
                drain(g, par)
                wb(g, par)
            return carry

        lax.fori_loop(0, (NG - 2) // 2, loop, 0)
        wb_wait(NG - 3, 1)
        issue(NG - 1, 1)
        drain(NG - 2, 0)
        wb(NG - 2, 0)
        drain(NG - 1, 1)
        wb(NG - 1, 1)
        wb_wait(NG - 2, 0)
        wb_wait(NG - 1, 1)


def _cat_gather(t_c, t_x, ix_c, ix_x):
    mesh = plsc.VectorSubcoreMesh(core_axis_name="c", subcore_axis_name="s")
    fn = functools.partial(
        pl.kernel,
        mesh=mesh,
        out_type=(jax.ShapeDtypeStruct((N4, 128), jnp.float32),
                  jax.ShapeDtypeStruct((N4, 128), jnp.float32)),
        scratch_types=[
            pltpu.VMEM((K, CH), jnp.int32),
            pltpu.VMEM((K, CH), jnp.int32),
            pltpu.VMEM_SHARED((V_CAT1, D), jnp.float32),
            pltpu.VMEM_SHARED((3, D), jnp.float32),
            pltpu.VMEM((GR, D), jnp.float32),
            pltpu.VMEM((GR, D), jnp.float32),
            pltpu.SemaphoreType.DMA,
            pltpu.SemaphoreType.DMA,
            pltpu.SemaphoreType.DMA,
            pltpu.SemaphoreType.DMA,
        ],
        compiler_params=pltpu.CompilerParams(
            use_tc_tiling_on_sc=False, needs_layout_passes=False),
    )(_cat_body)
    return fn(t_c, t_x, ix_c, ix_x)


BLK4 = 1024


def _mm_body(eu_ref, ei_ref, ec_ref, ex_ref, w_ref, b_ref, o_ref):
    acc = jnp.dot(eu_ref[...], w_ref[0], preferred_element_type=jnp.float32)
    acc = acc + jnp.dot(ei_ref[...], w_ref[1],
                        preferred_element_type=jnp.float32)
    acc = acc + jnp.dot(ec_ref[...], w_ref[2],
                        preferred_element_type=jnp.float32)
    acc = acc + jnp.dot(ex_ref[...], w_ref[3],
                        preferred_element_type=jnp.float32)
    for j in range(4):
        o_ref[j] = acc[:, j * 128:(j + 1) * 128] + b_ref[...]


def _project(eu, ei, ec, ex, W4, b1):
    espec = pl.BlockSpec((BLK4, 128), lambda i: (i, 0))
    return pl.pallas_call(
        _mm_body,
        grid=(N4 // BLK4,),
        in_specs=[
            espec, espec, espec, espec,
            pl.BlockSpec((4, 128, 512), lambda i: (0, 0, 0)),
            pl.BlockSpec((1, 128), lambda i: (0, 0)),
        ],
        out_specs=pl.BlockSpec((4, BLK4, 128), lambda i: (0, i, 0)),
        out_shape=jax.ShapeDtypeStruct((4, N4, 128), jnp.float32),
    )(eu, ei, ec, ex, W4, b1)


def _kron4(Wf):
    return (jnp.eye(4, dtype=jnp.float32)[:, None, :, None]
            * Wf[None, :, None, :]).reshape(4 * D, 4 * 128)


def kernel(user, item, category, interaction, emb_user, emb_item,
           emb_category, emb_interaction, W, b):
    def remap(v):
        v = v.astype(jnp.int32)
        u = v % TBLK
        return (v // TBLK) * TBLK + 4 * (u % (TBLK // 4)) + u // (TBLK // 4)

    ix_u = remap(user).T.reshape(NW, K, CH)
    ix_i = remap(item).T.reshape(NW, K, CH)
    ix_c = category.T.reshape(NW, K, CH).astype(jnp.int32)
    ix_x = interaction.T.reshape(NW, K, CH).astype(jnp.int32)
    eye = jnp.eye(128, dtype=jnp.float32)
    e_c, e_x = _cat_gather(emb_category, emb_interaction, ix_c, ix_x)
    tab_i = _to_rowmajor(emb_item.T, eye)
    e_i = _stream_field(tab_i, ix_i)
    tab_u = _to_rowmajor(emb_user.T, eye)
    e_u = _stream_field(tab_u, ix_u)
    W4 = jnp.stack([_kron4(W[f * D:(f + 1) * D, :]) for f in range(4)])
    out = _project(e_u, e_i, e_c, e_x, W4, b.reshape(1, 128))
    return jnp.transpose(out.reshape(L, B, 128), (1, 0, 2))


# speedup vs baseline: 25.7505x; 1.0322x over previous
import functools

import jax
import jax.numpy as jnp
from jax import lax
from jax.experimental import pallas as pl
from jax.experimental.pallas import tpu as pltpu
from jax.experimental.pallas import tpu_sc as plsc

B, L, D = 4096, 50, 32
N = B * L
N4 = N // 4
NC, NS = 2, 16
NW = NC * NS
PW = N // NW
CH = 128
CPG = 5
GR = CPG * CH
NG = PW // GR
K = PW // CH
V_CAT1 = 1001
CPAD = D + 1
TBLK = 16384


def _tr_body(t_ref, eye_ref, o_ref):
    x = t_ref[...]
    q = TBLK // 4
    x4 = jnp.concatenate(
        [x[:, 0:q], x[:, q:2 * q], x[:, 2 * q:3 * q], x[:, 3 * q:4 * q]],
        axis=0)
    o_ref[...] = lax.dot_general(x4, eye_ref[...], (((0,), (0,)), ((), ())),
                                 preferred_element_type=jnp.float32)


def _to_rowmajor(tabT, eye):
    V = tabT.shape[1]
    nb = (V + TBLK - 1) // TBLK
    out = pl.pallas_call(
        _tr_body,
        grid=(nb,),
        in_specs=[pl.BlockSpec((D, TBLK), lambda i: (0, i)),
                  pl.BlockSpec((128, 128), lambda i: (0, 0))],
        out_specs=pl.BlockSpec((TBLK // 4, 128), lambda i: (i, 0)),
        out_shape=jax.ShapeDtypeStruct((nb * TBLK // 4, 128), jnp.float32),
        compiler_params=pltpu.CompilerParams(
            fuse_transposed_lhs_in_matmul=True),
    )(tabT, eye)
    return out.reshape(nb * TBLK, D)


def _stream_body(tab, ix, out_e, idx_v, rows0, rows1, sg0, sg1, sw0, sw1):
    wid = lax.axis_index("s") * NC + lax.axis_index("c")
    jcol = (wid // 8) * D
    r0 = (wid % 8) * PW
    pltpu.sync_copy(ix.at[wid], idx_v)
    rows = (rows0, rows1)
    sg = (sg0, sg1)
    sw = (sw0, sw1)

    def issue(g, par):
        for t in range(CPG):
            pltpu.async_copy(tab.at[idx_v.at[g * CPG + t]],
                             rows[par].at[pl.ds(t * CH, CH), :], sg[par])

    def drain(g, par):
        for t in range(CPG):
            pltpu.make_async_copy(tab.at[idx_v.at[g * CPG + t]],
                                  rows[par].at[pl.ds(t * CH, CH), :],
                                  sg[par]).wait()

    def wb(g, par):
        pltpu.async_copy(
            rows[par],
            out_e.at[pl.ds(r0 + g * GR, GR), pl.ds(jcol, D)], sw[par])

    def wb_wait(g, par):
        pltpu.make_async_copy(
            rows[par],
            out_e.at[pl.ds(r0 + g * GR, GR), pl.ds(jcol, D)], sw[par]).wait()

    issue(0, 0)

    def loop(g2, carry):
        for par in range(2):
            g = 2 * g2 + par
            if par == 0:
                @pl.when(g2 > 0)
                def _(g=g):
                    wb_wait(g - 1, 1)
            else:
                wb_wait(g - 1, 0)
            issue(g + 1, 1 - par)
            drain(g, par)
            wb(g, par)
        return carry

    lax.fori_loop(0, (NG - 2) // 2, loop, 0)
    wb_wait(NG - 3, 1)
    issue(NG - 1, 1)
    drain(NG - 2, 0)
    wb(NG - 2, 0)
    drain(NG - 1, 1)
    wb(NG - 1, 1)
    wb_wait(NG - 2, 0)
    wb_wait(NG - 1, 1)


def _stream_field(tab, ix):
    mesh = plsc.VectorSubcoreMesh(core_axis_name="c", subcore_axis_name="s")
    fn = functools.partial(
        pl.kernel,
        mesh=mesh,
        out_type=jax.ShapeDtypeStruct((N4, 128), jnp.float32),
        scratch_types=[
            pltpu.VMEM((K, CH), jnp.int32),
            pltpu.VMEM((GR, D), jnp.float32),
            pltpu.VMEM((GR, D), jnp.float32),
            pltpu.SemaphoreType.DMA,
            pltpu.SemaphoreType.DMA,
            pltpu.SemaphoreType.DMA,
            pltpu.SemaphoreType.DMA,
        ],
        compiler_params=pltpu.CompilerParams(
            use_tc_tiling_on_sc=False, needs_layout_passes=False),
    )(_stream_body)
    return fn(tab, ix)


def _cat_body(t_c, t_x, ix_c, ix_x, e_c, e_x, idx_c, idx_x,
              cat_v, int_v, rows0, rows1, sg0, sg1, sw0, sw1):
    wid = lax.axis_index("s") * NC + lax.axis_index("c")
    jcol = (wid // 8) * D
    r0 = (wid % 8) * PW
    pltpu.sync_copy(ix_c.at[wid], idx_c)
    pltpu.sync_copy(ix_x.at[wid], idx_x)

    @pl.when(lax.axis_index("s") == 0)
    def _():
        pltpu.sync_copy(t_c, cat_v)
        pltpu.sync_copy(t_x, int_v)
    plsc.subcore_barrier()
    rows = (rows0, rows1)
    sg = (sg0, sg1)
    sw = (sw0, sw1)

    for tab_v, idx_v, out_e in ((cat_v, idx_c, e_c), (int_v, idx_x, e_x)):
        def issue(g, par, tab_v=tab_v, idx_v=idx_v):
            for t in range(CPG):
                pltpu.async_copy(tab_v.at[idx_v.at[g * CPG + t]],
                                 rows[par].at[pl.ds(t * CH, CH), :], sg[par])

        def drain(g, par, tab_v=tab_v, idx_v=idx_v):
            for t in range(CPG):
                pltpu.make_async_copy(tab_v.at[idx_v.at[g * CPG + t]],
                                      rows[par].at[pl.ds(t * CH, CH), :],
                                      sg[par]).wait()

        def wb(g, par, out_e=out_e):
            pltpu.async_copy(
                rows[par],
                out_e.at[pl.ds(r0 + g * GR, GR), pl.ds(jcol, D)], sw[par])

        def wb_wait(g, par, out_e=out_e):
            pltpu.make_async_copy(
                rows[par],
                out_e.at[pl.ds(r0 + g * GR, GR), pl.ds(jcol, D)],
                sw[par]).wait()

        issue(0, 0)

        def loop(g2, carry):
            for par in range(2):
                g = 2 * g2 + par
                if par == 0:
                    @pl.when(g2 > 0)
                    def _(g=g):
                        wb_wait(g - 1, 1)
                else:
                    wb_wait(g - 1, 0)
                issue(g + 1, 1 - par)
                drain(g, par)
                wb(g, par)
            return carry

        lax.fori_loop(0, (NG - 2) // 2, loop, 0)
        wb_wait(NG - 3, 1)
        issue(NG - 1, 1)
        drain(NG - 2, 0)
        wb(NG - 2, 0)
        drain(NG - 1, 1)
        wb(NG - 1, 1)
        wb_wait(NG - 2, 0)
        wb_wait(NG - 1, 1)


def _cat_gather(t_c, t_x, ix_c, ix_x):
    mesh = plsc.VectorSubcoreMesh(core_axis_name="c", subcore_axis_name="s")
    fn = functools.partial(
        pl.kernel,
        mesh=mesh,
        out_type=(jax.ShapeDtypeStruct((N4, 128), jnp.float32),
                  jax.ShapeDtypeStruct((N4, 128), jnp.float32)),
        scratch_types=[
            pltpu.VMEM((K, CH), jnp.int32),
            pltpu.VMEM((K, CH), jnp.int32),
            pltpu.VMEM_SHARED((V_CAT1, D), jnp.float32),
            pltpu.VMEM_SHARED((3, D), jnp.float32),
            pltpu.VMEM((GR, D), jnp.float32),
            pltpu.VMEM((GR, D), jnp.float32),
            pltpu.SemaphoreType.DMA,
            pltpu.SemaphoreType.DMA,
            pltpu.SemaphoreType.DMA,
            pltpu.SemaphoreType.DMA,
        ],
        compiler_params=pltpu.CompilerParams(
            use_tc_tiling_on_sc=False, needs_layout_passes=False),
    )(_cat_body)
    return fn(t_c, t_x, ix_c, ix_x)


BLK4 = 1024


def _mm_body(eu_ref, ei_ref, ec_ref, ex_ref, w_ref, b_ref, o_ref):
    acc = jnp.dot(eu_ref[...], w_ref[0], preferred_element_type=jnp.float32)
    acc = acc + jnp.dot(ei_ref[...], w_ref[1],
                        preferred_element_type=jnp.float32)
    acc = acc + jnp.dot(ec_ref[...], w_ref[2],
                        preferred_element_type=jnp.float32)
    acc = acc + jnp.dot(ex_ref[...], w_ref[3],
                        preferred_element_type=jnp.float32)
    for j in range(4):
        o_ref[j] = acc[:, j * 128:(j + 1) * 128] + b_ref[...]


def _project(eu, ei, ec, ex, W4, b1):
    espec = pl.BlockSpec((BLK4, 128), lambda i: (i, 0))
    return pl.pallas_call(
        _mm_body,
        grid=(N4 // BLK4,),
        in_specs=[
            espec, espec, espec, espec,
            pl.BlockSpec((4, 128, 512), lambda i: (0, 0, 0)),
            pl.BlockSpec((1, 128), lambda i: (0, 0)),
        ],
        out_specs=pl.BlockSpec((4, BLK4, 128), lambda i: (0, i, 0)),
        out_shape=jax.ShapeDtypeStruct((4, N4, 128), jnp.float32),
    )(eu, ei, ec, ex, W4, b1)


def _kron4(Wf):
    return (jnp.eye(4, dtype=jnp.float32)[:, None, :, None]
            * Wf[None, :, None, :]).reshape(4 * D, 4 * 128)


def kernel(user, item, category, interaction, emb_user, emb_item,
           emb_category, emb_interaction, W, b):
    def remap(v):
        v = v.astype(jnp.int32)
        u = v % TBLK
        return (v // TBLK) * TBLK + 4 * (u % (TBLK // 4)) + u // (TBLK // 4)

    ix_u = remap(user).T.reshape(NW, K, CH)
    ix_i = remap(item).T.reshape(NW, K, CH)
    ix_c = category.T.reshape(NW, K, CH).astype(jnp.int32)
    ix_x = interaction.T.reshape(NW, K, CH).astype(jnp.int32)
    eye = jnp.eye(128, dtype=jnp.float32)
    e_c, e_x = _cat_gather(emb_category, emb_interaction, ix_c, ix_x)
    tab_i = _to_rowmajor(emb_item.T, eye)
    e_i = _stream_field(tab_i, ix_i)
    tabT_u, _ = lax.optimization_barrier((emb_user.T, e_c))
    tab_u = _to_rowmajor(tabT_u, eye)
    e_u = _stream_field(tab_u, ix_u)
    W4 = jnp.stack([_kron4(W[f * D:(f + 1) * D, :]) for f in range(4)])
    out = _project(e_u, e_i, e_c, e_x, W4, b.reshape(1, 128))
    return jnp.transpose(out.reshape(L, B, 128), (1, 0, 2))
